# baseline (speedup 1.0000x reference)
import functools

import jax
import jax.numpy as jnp
from jax.experimental import pallas as pl
from jax.experimental.pallas import tpu as pltpu

_CH = 256
_N_BUF = 8
_W1_CH = 256


def _router_body(x_hbm, w1_hbm, b1_ref, w2_ref, b2_ref,
                 rw_ref, idx_ref,
                 acc_ref, bufs_ref, w1_vmem, x_sems, w1_sems,
                 *, n_chunks, n_w1, b, blk_per_b, s_total):
    def x_copy(c):
        return pltpu.make_async_copy(
            x_hbm.at[pl.ds(c * _CH, _CH), :],
            bufs_ref.at[c % _N_BUF],
            x_sems.at[c % _N_BUF])

    def w1_copy(j):
        return pltpu.make_async_copy(
            w1_hbm.at[pl.ds(j * _W1_CH, _W1_CH), :],
            w1_vmem.at[pl.ds(j * _W1_CH, _W1_CH), :],
            w1_sems.at[j])

    for c in range(_N_BUF):
        x_copy(c).start()
    for j in range(n_w1):
        w1_copy(j).start()

    ones = jnp.ones((1, _CH), jnp.float32)
    for c in range(n_chunks):
        x_copy(c).wait()
        acc_ref[pl.ds(c, 1), :] = jnp.dot(
            ones, bufs_ref[c % _N_BUF],
            preferred_element_type=jnp.float32)
        if c + _N_BUF < n_chunks:
            x_copy(c + _N_BUF).start()

    for j in range(n_w1):
        w1_copy(j).wait()

    nb = acc_ref.shape[0]
    rows = jax.lax.broadcasted_iota(jnp.int32, (b, nb), 0)
    cols = jax.lax.broadcasted_iota(jnp.int32, (b, nb), 1)
    sel = (cols // blk_per_b == rows).astype(jnp.float32) * (1.0 / s_total)
    fv = jnp.dot(sel, acc_ref[...], preferred_element_type=jnp.float32)
    h = jnp.dot(fv, w1_vmem[...], preferred_element_type=jnp.float32)
    h = jnp.maximum(h + b1_ref[...], 0.0)
    logits = jnp.dot(h, w2_ref[...], preferred_element_type=jnp.float32)
    logits = logits + b2_ref[...]
    m = jnp.max(logits, axis=-1, keepdims=True)
    e = jnp.exp(logits - m)
    w = e / jnp.sum(e, axis=-1, keepdims=True)
    rw_ref[...] = w
    ncols = w.shape[-1]
    ids = jax.lax.broadcasted_iota(jnp.int32, w.shape, 1)
    m1 = jnp.max(w, axis=-1, keepdims=True)
    i1 = jnp.min(jnp.where(w == m1, ids, ncols), axis=-1, keepdims=True)
    wm = jnp.where(ids == i1, -jnp.inf, w)
    m2 = jnp.max(wm, axis=-1, keepdims=True)
    i2 = jnp.min(jnp.where(wm == m2, ids, ncols), axis=-1, keepdims=True)
    col = jax.lax.broadcasted_iota(jnp.int32, idx_ref.shape, 1)
    idx_ref[...] = jnp.where(col == 0, i1, i2)


@jax.jit
def kernel(hidden_states, W1, b1, W2, b2):
    B, S, H = hidden_states.shape
    E = W2.shape[1]
    x2d = hidden_states.reshape(B * S, H)
    n_chunks = (B * S) // _CH
    n_w1 = H // _W1_CH
    blk_per_b = S // _CH

    b1r = b1.reshape(1, H)
    b2r = b2.reshape(1, E)

    body = functools.partial(
        _router_body, n_chunks=n_chunks, n_w1=n_w1, b=B,
        blk_per_b=blk_per_b, s_total=S)

    rw, idx = pl.pallas_call(
        body,
        grid=(1,),
        in_specs=[
            pl.BlockSpec(memory_space=pl.ANY),
            pl.BlockSpec(memory_space=pl.ANY),
            pl.BlockSpec((1, H), lambda i: (0, 0)),
            pl.BlockSpec((H, E), lambda i: (0, 0)),
            pl.BlockSpec((1, E), lambda i: (0, 0)),
        ],
        out_specs=[
            pl.BlockSpec((B, E), lambda i: (0, 0)),
            pl.BlockSpec((B, 2), lambda i: (0, 0)),
        ],
        out_shape=[
            jax.ShapeDtypeStruct((B, E), jnp.float32),
            jax.ShapeDtypeStruct((B, 2), jnp.int32),
        ],
        scratch_shapes=[
            pltpu.VMEM((n_chunks, H), jnp.float32),
            pltpu.VMEM((_N_BUF, _CH, H), jnp.float32),
            pltpu.VMEM((H, H), jnp.float32),
            pltpu.SemaphoreType.DMA((_N_BUF,)),
            pltpu.SemaphoreType.DMA((n_w1,)),
        ],
    )(x2d, W1, b1r, W2, b2r)
    return rw, idx
